# baseline (speedup 1.0000x reference)
import jax
import jax.numpy as jnp
from jax.experimental import pallas as pl

NUM_GROUP = 1024
GROUP_SIZE = 64


def _sub_kernel(nbr_ref, ctr_ref, out_ref):
    out_ref[...] = nbr_ref[...] - ctr_ref[...]


def kernel(xyz, sample_idx):
    B, N, _ = xyz.shape
    center_idx = sample_idx.astype(jnp.int32).reshape(1, -1)
    center = jnp.take(xyz, center_idx[0], axis=1)
    center = jnp.where(jnp.isnan(center), jnp.zeros_like(center), center)
    d = (jnp.sum(center ** 2, axis=-1)[:, :, None]
         + jnp.sum(xyz ** 2, axis=-1)[:, None, :]
         - 2.0 * jnp.einsum('bgd,bnd->bgn', center, xyz))
    _, idx = jax.lax.top_k(-d, GROUP_SIZE)
    ori_idx = idx
    flat_idx = idx.reshape(-1)
    neighborhood = xyz.reshape(B * N, 3)[flat_idx, :]
    neighborhood = neighborhood.reshape(B, NUM_GROUP, GROUP_SIZE, 3)
    ctr = jnp.broadcast_to(center[:, :, None, :], neighborhood.shape)
    n2 = neighborhood.reshape(NUM_GROUP, GROUP_SIZE * 3)
    c2 = ctr.reshape(NUM_GROUP, GROUP_SIZE * 3)
    out = pl.pallas_call(
        _sub_kernel,
        out_shape=jax.ShapeDtypeStruct(n2.shape, n2.dtype),
    )(n2, c2)
    return (out.reshape(B, NUM_GROUP, GROUP_SIZE, 3), center, ori_idx, center_idx)


# speedup vs baseline: 6.9686x; 6.9686x over previous
import functools

import jax
import jax.numpy as jnp
import numpy as np
from jax import lax
from jax.experimental import pallas as pl
from jax.experimental.pallas import tpu as pltpu
from jax.experimental.pallas import tpu_sc as plsc

N = 16384
G = 1024
K = 64
L = 16
NC = 2
NS = 16
NW = NC * NS
GPW = G // NW
GRP = 16
NGRP = N // (L * GRP)
PEND = 288
PEND_ALLOC = 384

_INF = np.float32(np.inf)
_BIGI = np.int32(0x7FFFFFFF)


def _rne_bf16(v):
    u = lax.bitcast_convert_type(v, jnp.uint32)
    r = (u + np.uint32(0x7FFF) + ((u >> 16) & np.uint32(1))) & np.uint32(0xFFFF0000)
    return lax.bitcast_convert_type(r, jnp.float32)


def _splat(ref, j):
    return plsc.load_gather(ref, [jnp.full((L,), j, jnp.int32)])


def _cascade(accd, acci, rd, ri):
    cd, ci = rd, ri
    for b in range(4):
        bd = accd[pl.ds(b * L, L)]
        bi = acci[pl.ds(b * L, L)]
        rcd = lax.rev(cd, (0,))
        rci = lax.rev(ci, (0,))
        less = (bd < rcd) | ((bd == rcd) & (bi < rci))
        lod = jnp.where(less, bd, rcd)
        loi = jnp.where(less, bi, rci)
        lod, loi = plsc.sort_key_val(lod, loi)
        accd[pl.ds(b * L, L)] = lod
        acci[pl.ds(b * L, L)] = loi
        if b < 3:
            hid = jnp.where(less, rcd, bd)
            hii = jnp.where(less, rci, bi)
            cd, ci = plsc.sort_key_val(hid, hii)


def _sc_group_kernel(x_hbm, y_hbm, z_hbm, sidx_hbm,
                     nbh_hbm, ctr_hbm, oidx_hbm,
                     xv, yv, zv, qv, xb, yb, zb, sidx_v,
                     accd, acci, pendd, pendi,
                     nbh_buf, ctr_buf, oidx_buf):
    wid = lax.axis_index("s") * NC + lax.axis_index("c")

    pltpu.sync_copy(x_hbm, xv)
    pltpu.sync_copy(y_hbm, yv)
    pltpu.sync_copy(z_hbm, zv)
    pltpu.sync_copy(sidx_hbm.at[pl.ds(wid * GPW, GPW)], sidx_v.at[pl.ds(0, GPW)])

    ii = lax.iota(jnp.int32, L)

    def _qbody(i, _):
        b = i * L
        xc = xv[pl.ds(b, L)]
        yc = yv[pl.ds(b, L)]
        zc = zv[pl.ds(b, L)]
        qv[pl.ds(b, L)] = (xc * xc + yc * yc) + zc * zc
        xb[pl.ds(b, L)] = _rne_bf16(xc)
        yb[pl.ds(b, L)] = _rne_bf16(yc)
        zb[pl.ds(b, L)] = _rne_bf16(zc)
        return 0

    lax.fori_loop(0, N // L, _qbody, 0)

    def _center(j, _):
        sj = _splat(sidx_v, j)
        cx = plsc.load_gather(xv, [sj])
        cy = plsc.load_gather(yv, [sj])
        cz = plsc.load_gather(zv, [sj])
        cx = jnp.where(cx != cx, np.float32(0), cx)
        cy = jnp.where(cy != cy, np.float32(0), cy)
        cz = jnp.where(cz != cz, np.float32(0), cz)
        cq = (cx * cx + cz * cz) + cy * cy
        cbx = _rne_bf16(cx)
        cby = _rne_bf16(cy)
        cbz = _rne_bf16(cz)

        for b in range(4):
            accd[pl.ds(b * L, L)] = jnp.full((L,), _INF)
            acci[pl.ds(b * L, L)] = jnp.full((L,), _BIGI)

        def _drain_cond(carry):
            cntv, _ = carry
            return jnp.any(cntv >= L)

        def _drain_body(carry):
            cntv, tv = carry
            rd = pendd[pl.ds(0, L)]
            ri = pendi[pl.ds(0, L)]
            rd, ri = plsc.sort_key_val(rd, ri)
            _cascade(accd, acci, rd, ri)
            for s in range(PEND // L - 1):
                pendd[pl.ds(s * L, L)] = pendd[pl.ds((s + 1) * L, L)]
                pendi[pl.ds(s * L, L)] = pendi[pl.ds((s + 1) * L, L)]
            tv = _splat(accd, 63)
            return cntv - L, tv

        def _group(g, carry):
            cntv, tv = carry
            for c in range(GRP):
                base = (g * GRP + c) * L
                xc = xb[pl.ds(base, L)]
                yc = yb[pl.ds(base, L)]
                zc = zb[pl.ds(base, L)]
                qc = qv[pl.ds(base, L)]
                dot = (cbx * xc + cby * yc) + cbz * zc
                d = (cq + qc) - np.float32(2) * dot
                m = d <= tv
                mi = m.astype(jnp.int32)
                pos = (cntv + plsc.cumsum(mi)) - 1
                plsc.store_scatter(pendd, [pos], d, mask=m)
                plsc.store_scatter(pendi, [pos], ii + base, mask=m)
                cntv = cntv + plsc.all_reduce_population_count(m)
            return lax.while_loop(_drain_cond, _drain_body, (cntv, tv))

        cnt0 = jnp.zeros((L,), jnp.int32)
        tv0 = jnp.full((L,), _INF)
        cntv, tv = lax.fori_loop(0, NGRP, _group, (cnt0, tv0))

        rd = pendd[pl.ds(0, L)]
        ri = pendi[pl.ds(0, L)]
        valid = ii < cntv
        rd = jnp.where(valid, rd, _INF)
        ri = jnp.where(valid, ri, _BIGI)
        rd, ri = plsc.sort_key_val(rd, ri)
        _cascade(accd, acci, rd, ri)

        for b in range(4):
            ids0 = acci[pl.ds(b * L, L)]
            dsb = accd[pl.ds(b * L, L)]
            nxt = jnp.minimum(b * L + ii + 1, 63)
            prv = jnp.maximum(b * L + ii - 1, 0)
            dn = plsc.load_gather(accd, [nxt])
            inx = plsc.load_gather(acci, [nxt])
            dp = plsc.load_gather(accd, [prv])
            ip = plsc.load_gather(acci, [prv])
            ids = jnp.where((dsb == dn) & (ids0 > inx), inx, ids0)
            ids = jnp.where((dsb == dp) & (ids0 < ip), ip, ids)
            gx = plsc.load_gather(xv, [ids]) - cx
            gy = plsc.load_gather(yv, [ids]) - cy
            gz = plsc.load_gather(zv, [ids]) - cz
            pos = (j * K + b * L + ii) * 3
            plsc.store_scatter(nbh_buf, [pos], gx)
            plsc.store_scatter(nbh_buf, [pos + 1], gy)
            plsc.store_scatter(nbh_buf, [pos + 2], gz)
            oidx_buf[pl.ds(j * K + b * L, L)] = ids
        lane0 = ii == 0
        plsc.store_scatter(ctr_buf, [jnp.full((L,), 3 * j, jnp.int32)], cx,
                           mask=lane0)
        plsc.store_scatter(ctr_buf, [jnp.full((L,), 3 * j + 1, jnp.int32)], cy,
                           mask=lane0)
        plsc.store_scatter(ctr_buf, [jnp.full((L,), 3 * j + 2, jnp.int32)], cz,
                           mask=lane0)
        return 0

    lax.fori_loop(0, GPW, _center, 0)

    pltpu.sync_copy(nbh_buf, nbh_hbm.at[pl.ds(wid * GPW * K * 3, GPW * K * 3)])
    pltpu.sync_copy(ctr_buf.at[pl.ds(0, GPW * 3)], ctr_hbm.at[pl.ds(wid * GPW * 3, GPW * 3)])
    pltpu.sync_copy(oidx_buf, oidx_hbm.at[pl.ds(wid * GPW * K, GPW * K)])


_sc_call = functools.partial(
    pl.kernel,
    out_type=(
        jax.ShapeDtypeStruct((G * K * 3,), jnp.float32),
        jax.ShapeDtypeStruct((G * 3,), jnp.float32),
        jax.ShapeDtypeStruct((G * K,), jnp.int32),
    ),
    mesh=plsc.VectorSubcoreMesh(core_axis_name="c", subcore_axis_name="s",
                                num_cores=NC, num_subcores=NS),
    compiler_params=pltpu.CompilerParams(needs_layout_passes=False),
    scratch_types=[
        pltpu.VMEM((N,), jnp.float32),
        pltpu.VMEM((N,), jnp.float32),
        pltpu.VMEM((N,), jnp.float32),
        pltpu.VMEM((N,), jnp.float32),
        pltpu.VMEM((N,), jnp.float32),
        pltpu.VMEM((N,), jnp.float32),
        pltpu.VMEM((N,), jnp.float32),
        pltpu.VMEM((128,), jnp.int32),
        pltpu.VMEM((128,), jnp.float32),
        pltpu.VMEM((128,), jnp.int32),
        pltpu.VMEM((PEND_ALLOC,), jnp.float32),
        pltpu.VMEM((PEND_ALLOC,), jnp.int32),
        pltpu.VMEM((GPW * K * 3,), jnp.float32),
        pltpu.VMEM((128,), jnp.float32),
        pltpu.VMEM((GPW * K,), jnp.int32),
    ],
)(_sc_group_kernel)


def kernel(xyz, sample_idx):
    B = xyz.shape[0]
    pts = xyz[0]
    x = pts[:, 0]
    y = pts[:, 1]
    z = pts[:, 2]
    sidx = sample_idx.astype(jnp.int32)
    nbh, ctr, oidx = _sc_call(x, y, z, sidx)
    center_idx = sidx.reshape(1, -1)
    return (
        nbh.reshape(B, G, K, 3),
        ctr.reshape(B, G, 3),
        oidx.reshape(B, G, K),
        center_idx,
    )
